# baseline (speedup 1.0000x reference)
import functools

import jax
import jax.numpy as jnp
from jax import lax
from jax.experimental import pallas as pl
from jax.experimental.pallas import tpu as pltpu
from jax.experimental.pallas import tpu_sc as plsc

_INFO = plsc.get_sparse_core_info()
_NC = _INFO.num_cores
_NS = _INFO.num_subcores
_NW = _NC * _NS

_CHUNK = 40


def _gather_rows(n_rows: int, d: int, d_pad: int):
    b_per_w = n_rows // _NW
    n_chunks = b_per_w // _CHUNK
    mesh = plsc.VectorSubcoreMesh(core_axis_name="c", subcore_axis_name="s")

    @functools.partial(
        pl.kernel,
        mesh=mesh,
        out_type=jax.ShapeDtypeStruct((n_rows, d_pad), jnp.float32),
        scratch_types=[
            pltpu.VMEM((b_per_w,), jnp.int32),
            pltpu.VMEM((2, _CHUNK, d_pad), jnp.float32),
            pltpu.SemaphoreType.DMA,
            pltpu.SemaphoreType.DMA,
        ],
    )
    def k(idx_hbm, table_hbm, out_hbm, idx_v, rows_v, gsem, osem):
        wid = lax.axis_index("s") * _NC + lax.axis_index("c")
        base = wid * b_per_w
        pltpu.sync_copy(idx_hbm.at[pl.ds(base, b_per_w)], idx_v)

        def gather(i, buf):
            return pltpu.async_copy(
                table_hbm.at[idx_v.at[pl.ds(i * _CHUNK, _CHUNK)]],
                rows_v.at[buf],
                gsem,
            )

        def writeout(i, buf):
            return pltpu.async_copy(
                rows_v.at[buf],
                out_hbm.at[pl.ds(base + i * _CHUNK, _CHUNK)],
                osem,
            )

        g = [None, None]
        o = [None, None]
        g[0] = gather(0, 0)
        for i in range(n_chunks):
            buf = i % 2
            nxt = (i + 1) % 2
            if i + 1 < n_chunks:
                if o[nxt] is not None:
                    o[nxt].wait()
                g[nxt] = gather(i + 1, nxt)
            g[buf].wait()
            o[buf] = writeout(i, buf)
        o[(n_chunks - 1) % 2].wait()
        if n_chunks > 1:
            o[n_chunks % 2].wait()

    return k


def kernel(x, embedding_table):
    b, t = x.shape
    v, d = embedding_table.shape
    d_pad = (d + 127) // 128 * 128
    n = b * t
    idx = x.reshape(n).astype(jnp.int32)
    table_p = jnp.pad(embedding_table, ((0, 0), (0, d_pad - d)))
    out = _gather_rows(n, d, d_pad)(idx, table_p)
    return out[:, :d].reshape(b, t, d)


# speedup vs baseline: 1.5581x; 1.5581x over previous
import functools

import jax
import jax.numpy as jnp
from jax import lax
from jax.experimental import pallas as pl
from jax.experimental.pallas import tpu as pltpu
from jax.experimental.pallas import tpu_sc as plsc

_INFO = plsc.get_sparse_core_info()
_NC = _INFO.num_cores
_NS = _INFO.num_subcores
_NW = _NC * _NS

_TPAD = 24


def _gather_rows(nb: int, t: int, d_pad: int):
    b_per_w = nb // _NW
    mesh = plsc.VectorSubcoreMesh(core_axis_name="c", subcore_axis_name="s")

    @functools.partial(
        pl.kernel,
        mesh=mesh,
        out_type=jax.ShapeDtypeStruct((nb, t, d_pad), jnp.float32),
        scratch_types=[
            pltpu.VMEM((b_per_w * _TPAD,), jnp.int32),
            pltpu.VMEM((2, t, d_pad), jnp.float32),
            pltpu.SemaphoreType.DMA,
            pltpu.SemaphoreType.DMA,
        ],
    )
    def k(idx_hbm, table_hbm, out_hbm, idx_v, rows_v, gsem, osem):
        wid = lax.axis_index("s") * _NC + lax.axis_index("c")
        base_b = wid * b_per_w
        pltpu.sync_copy(
            idx_hbm.at[pl.ds(base_b * _TPAD, b_per_w * _TPAD)], idx_v
        )

        def gather(i, buf):
            return pltpu.async_copy(
                table_hbm.at[idx_v.at[pl.ds(i * _TPAD, t)]],
                rows_v.at[buf],
                gsem,
            )

        def writeout(i, buf):
            return pltpu.async_copy(
                rows_v.at[buf],
                out_hbm.at[base_b + i],
                osem,
            )

        def gather_wait():
            pltpu.make_async_copy(
                table_hbm.at[idx_v.at[pl.ds(0, t)]], rows_v.at[0], gsem
            ).wait()

        def write_wait():
            pltpu.make_async_copy(rows_v.at[0], out_hbm.at[base_b], osem).wait()

        gather(0, 0)
        gather(1, 1)
        gather_wait()
        writeout(0, 0)

        def body(i, carry):
            buf = lax.rem(i, 2)
            nxt = lax.rem(i + 1, 2)
            write_wait()
            gather(i + 1, nxt)
            gather_wait()
            writeout(i, buf)
            return carry

        lax.fori_loop(1, b_per_w - 1, body, 0)

        gather_wait()
        writeout(b_per_w - 1, lax.rem(b_per_w - 1, 2))
        write_wait()
        write_wait()

    return k


def kernel(x, embedding_table):
    b, t = x.shape
    v, d = embedding_table.shape
    d_pad = (d + 127) // 128 * 128
    idx = jnp.pad(x.astype(jnp.int32), ((0, 0), (0, _TPAD - t))).reshape(-1)
    table_p = jnp.pad(embedding_table, ((0, 0), (0, d_pad - d)))
    out = _gather_rows(b, t, d_pad)(idx, table_p)
    return out[:, :, :d]
